# Initial kernel scaffold; baseline (speedup 1.0000x reference)
import jax
import jax.numpy as jnp
from jax.experimental import pallas as pl


def kernel(adj, features, weight, weight2, W1, b1, W2, b2):
    raise NotImplementedError("write your pallas kernel here")


# speedup vs baseline: 1.0144x; 1.0144x over previous
import jax
import jax.numpy as jnp
from jax.experimental import pallas as pl

_BM = 400


def _pass1(adj_ref, featk_ref, featm_ref, w_ref, w1t_ref, b1_ref,
           w2t_ref, b2_ref, conv1_ref, self_ref):
    a = adj_ref[...].astype(jnp.bfloat16)
    fk = featk_ref[...].astype(jnp.bfloat16)
    g = jnp.dot(a, fk, preferred_element_type=jnp.float32)
    conv1_ref[...] = jnp.maximum(
        jnp.dot(g, w_ref[...], preferred_element_type=jnp.float32), 0.0)
    h = jnp.maximum(
        jnp.dot(featm_ref[...], w1t_ref[...],
                preferred_element_type=jnp.float32) + b1_ref[...], 0.0)
    self_ref[...] = jnp.dot(
        h, w2t_ref[...], preferred_element_type=jnp.float32) + b2_ref[...]


def _pass2(adj_ref, c1k_ref, c1m_ref, selfm_ref, w2_ref, out_ref):
    a = adj_ref[...].astype(jnp.bfloat16)
    ck = c1k_ref[...].astype(jnp.bfloat16)
    h2 = jnp.dot(a, ck, preferred_element_type=jnp.float32)
    conv2 = jnp.dot(h2, w2_ref[...], preferred_element_type=jnp.float32)
    out_ref[:, 0:128] = selfm_ref[...]
    out_ref[:, 128:256] = c1m_ref[...]
    out_ref[:, 256:384] = conv2


def kernel(adj, features, weight, weight2, W1, b1, W2, b2):
    n, f = features.shape
    grid = (n // _BM,)
    b1r = b1.reshape(1, f)
    b2r = b2.reshape(1, f)
    w1t = W1.T
    w2t = W2.T

    full = lambda i: (0, 0)
    rows = lambda i: (i, 0)

    conv1, self_out = pl.pallas_call(
        _pass1,
        grid=grid,
        in_specs=[
            pl.BlockSpec((_BM, n), rows),
            pl.BlockSpec((n, f), full),
            pl.BlockSpec((_BM, f), rows),
            pl.BlockSpec((f, f), full),
            pl.BlockSpec((f, f), full),
            pl.BlockSpec((1, f), full),
            pl.BlockSpec((f, f), full),
            pl.BlockSpec((1, f), full),
        ],
        out_specs=[
            pl.BlockSpec((_BM, f), rows),
            pl.BlockSpec((_BM, f), rows),
        ],
        out_shape=[
            jax.ShapeDtypeStruct((n, f), jnp.float32),
            jax.ShapeDtypeStruct((n, f), jnp.float32),
        ],
    )(adj, features, features, weight, w1t, b1r, w2t, b2r)

    out = pl.pallas_call(
        _pass2,
        grid=grid,
        in_specs=[
            pl.BlockSpec((_BM, n), rows),
            pl.BlockSpec((n, f), full),
            pl.BlockSpec((_BM, f), rows),
            pl.BlockSpec((_BM, f), rows),
            pl.BlockSpec((f, f), full),
        ],
        out_specs=pl.BlockSpec((_BM, 3 * f), rows),
        out_shape=jax.ShapeDtypeStruct((n, 3 * f), jnp.float32),
    )(adj, conv1, conv1, self_out, weight2)
    return out
